# baseline (speedup 1.0000x reference)
import functools

import jax
import jax.numpy as jnp
from jax import lax
from jax.experimental import pallas as pl
from jax.experimental.pallas import tpu as pltpu
from jax.experimental.pallas import tpu_sc as plsc


def _build_sc_kernel(B, S, P, H, V):
    info = plsc.get_sparse_core_info()
    NC, NS, L = info.num_cores, info.num_subcores, info.num_lanes
    NW = NC * NS
    assert (B * S) % NW == 0
    TPW = (B * S) // NW
    WPR = NW // B
    assert S % WPR == 0 and TPW == S // WPR
    CH = 8
    NBUF = 12
    DEPTH = 8
    NCHUNK = TPW // CH
    IPW = P // WPR

    mesh = plsc.VectorSubcoreMesh(core_axis_name="c", subcore_axis_name="s")

    @functools.partial(
        pl.kernel,
        out_type=[
            jax.ShapeDtypeStruct((B, S, H), jnp.float32),
            jax.ShapeDtypeStruct((B, S), jnp.int32),
        ],
        mesh=mesh,
        scratch_types=[
            pltpu.VMEM((S,), jnp.int32),
            pltpu.VMEM((L,), jnp.int32),
            pltpu.VMEM((TPW,), jnp.int32),
        ] + [pltpu.VMEM((CH, H), jnp.float32) for _ in range(NBUF)]
          + [pltpu.VMEM((L, H), jnp.float32)]
          + [pltpu.VMEM((L,), jnp.int32)]
          + [pltpu.SemaphoreType.DMA for _ in range(2 * NBUF + 1)],
    )
    def body(ids_hbm, img_hbm, tid_hbm, w_hbm, out_hbm, mask_hbm,
             row_v, tid_v, mask_v, *rest):
        bufs = rest[:NBUF]
        ibuf = rest[NBUF]
        sidx_v = rest[NBUF + 1]
        gsems = rest[NBUF + 2:2 * NBUF + 2]
        osems = rest[2 * NBUF + 2:3 * NBUF + 2]
        sem = rest[3 * NBUF + 2]
        c = lax.axis_index("c")
        s = lax.axis_index("s")
        wid = c * NS + s
        b = wid // WPR
        kw = wid % WPR
        loc = kw * TPW
        t0 = b * S + loc

        pltpu.sync_copy(ids_hbm.at[b], row_v)
        pltpu.sync_copy(tid_hbm, tid_v)
        tidv = tid_v[...]

        gd = [None] * NBUF
        od = [None] * NBUF

        def fire_g(k):
            gd[k % NBUF] = pltpu.async_copy(
                w_hbm.at[row_v.at[pl.ds(loc + k * CH, CH)]],
                bufs[k % NBUF], gsems[k % NBUF])

        def fire_o(k):
            od[k % NBUF] = pltpu.async_copy(
                bufs[k % NBUF], out_hbm.at[b, pl.ds(loc + k * CH, CH)],
                osems[k % NBUF])

        for k in range(DEPTH):
            fire_g(k)

        def scan_body(i, acc):
            v = row_v[pl.ds(i * L, L)]
            posv = lax.iota(jnp.int32, L) + i * L
            return jnp.minimum(acc, jnp.where(v == tidv, posv, S))

        fpv = lax.fori_loop(0, S // L, scan_body,
                            jnp.full((L,), S, jnp.int32))
        dnums = lax.GatherDimensionNumbers(
            offset_dims=(), collapsed_slice_dims=(0,), start_index_map=(0,))
        for st in (1, 2, 4, 8):
            perm = (lax.iota(jnp.int32, L) ^ st)[:, None]
            fpv = jnp.minimum(
                fpv, lax.gather(fpv, perm, dnums, (1,),
                                mode=lax.GatherScatterMode.PROMISE_IN_BOUNDS))
        fp = jnp.squeeze(lax.slice(fpv, (0,), (1,)))
        valid = fp <= S - P

        iota = lax.iota(jnp.int32, L)
        one = jnp.full((L,), 1, jnp.int32)
        zero = jnp.full((L,), 0, jnp.int32)
        for j in range(TPW // L):
            v = row_v[pl.ds(loc + j * L, L)]
            mask_v[pl.ds(j * L, L)] = jnp.where(v != -100, one, zero)
        pltpu.sync_copy(mask_v, mask_hbm.at[b, pl.ds(loc, TPW)])

        o_pending = [False] * NCHUNK
        for k in range(NCHUNK):
            gd[k % NBUF].wait()
            fire_o(k)
            o_pending[k] = True
            nk = k + DEPTH
            if nk < NCHUNK:
                j = nk - NBUF
                if j >= 0:
                    od[j % NBUF].wait()
                    o_pending[j] = False
                fire_g(nk)
        for k in range(NCHUNK):
            if o_pending[k]:
                od[k % NBUF].wait()

        plsc.subcore_barrier()

        @pl.when(valid)
        def _image_overwrite():
            for h in range(IPW // L):
                pltpu.sync_copy(
                    img_hbm.at[b, pl.ds(kw * IPW + h * L, L)], ibuf)
                sidx_v[...] = fpv + kw * IPW + h * L + iota
                pltpu.async_copy(ibuf, out_hbm.at[b].at[sidx_v], sem).wait()

    return body


def kernel(input_ids, image_features, image_token_id, W):
    B, S = input_ids.shape
    _, P, H = image_features.shape
    V = W.shape[0]

    ids = input_ids.astype(jnp.int32)
    tid = jnp.full((16,), image_token_id, dtype=jnp.int32)

    sc = _build_sc_kernel(B, S, P, H, V)
    out, mask = sc(ids, image_features, tid, W)
    return out, mask


# speedup vs baseline: 1.0323x; 1.0323x over previous
import functools

import jax
import jax.numpy as jnp
from jax import lax
from jax.experimental import pallas as pl
from jax.experimental.pallas import tpu as pltpu
from jax.experimental.pallas import tpu_sc as plsc


def _build_sc_kernel(B, S, P, H, V):
    info = plsc.get_sparse_core_info()
    NC, NS, L = info.num_cores, info.num_subcores, info.num_lanes
    NW = NC * NS
    assert (B * S) % NW == 0
    TPW = (B * S) // NW
    WPR = NW // B
    assert S % WPR == 0 and TPW == S // WPR
    CH = 16
    NBUF = 6
    DEPTH = 5
    NCHUNK = TPW // CH
    IPW = P // WPR

    mesh = plsc.VectorSubcoreMesh(core_axis_name="c", subcore_axis_name="s")

    @functools.partial(
        pl.kernel,
        out_type=[
            jax.ShapeDtypeStruct((B, S, H), jnp.float32),
            jax.ShapeDtypeStruct((B, S), jnp.int32),
        ],
        mesh=mesh,
        scratch_types=[
            pltpu.VMEM((S,), jnp.int32),
            pltpu.VMEM((L,), jnp.int32),
            pltpu.VMEM((TPW,), jnp.int32),
        ] + [pltpu.VMEM((CH, H), jnp.float32) for _ in range(NBUF)]
          + [pltpu.VMEM((L, H), jnp.float32)]
          + [pltpu.VMEM((L,), jnp.int32)]
          + [pltpu.SemaphoreType.DMA for _ in range(2 * NBUF + 1)],
    )
    def body(ids_hbm, img_hbm, tid_hbm, w_hbm, out_hbm, mask_hbm,
             row_v, tid_v, mask_v, *rest):
        bufs = rest[:NBUF]
        ibuf = rest[NBUF]
        sidx_v = rest[NBUF + 1]
        gsems = rest[NBUF + 2:2 * NBUF + 2]
        osems = rest[2 * NBUF + 2:3 * NBUF + 2]
        sem = rest[3 * NBUF + 2]
        c = lax.axis_index("c")
        s = lax.axis_index("s")
        wid = c * NS + s
        b = wid // WPR
        kw = wid % WPR
        loc = kw * TPW
        t0 = b * S + loc

        pltpu.sync_copy(ids_hbm.at[b], row_v)
        pltpu.sync_copy(tid_hbm, tid_v)
        tidv = tid_v[...]

        gd = [None] * NBUF
        od = [None] * NBUF

        def fire_g(k):
            gd[k % NBUF] = pltpu.async_copy(
                w_hbm.at[row_v.at[pl.ds(loc + k * CH, CH)]],
                bufs[k % NBUF], gsems[k % NBUF])

        def fire_o(k):
            od[k % NBUF] = pltpu.async_copy(
                bufs[k % NBUF], out_hbm.at[b, pl.ds(loc + k * CH, CH)],
                osems[k % NBUF])

        for k in range(DEPTH):
            fire_g(k)

        def scan_body(i, acc):
            v = row_v[pl.ds(i * L, L)]
            posv = lax.iota(jnp.int32, L) + i * L
            return jnp.minimum(acc, jnp.where(v == tidv, posv, S))

        fpv = lax.fori_loop(0, S // L, scan_body,
                            jnp.full((L,), S, jnp.int32))
        dnums = lax.GatherDimensionNumbers(
            offset_dims=(), collapsed_slice_dims=(0,), start_index_map=(0,))
        for st in (1, 2, 4, 8):
            perm = (lax.iota(jnp.int32, L) ^ st)[:, None]
            fpv = jnp.minimum(
                fpv, lax.gather(fpv, perm, dnums, (1,),
                                mode=lax.GatherScatterMode.PROMISE_IN_BOUNDS))
        fp = jnp.squeeze(lax.slice(fpv, (0,), (1,)))
        valid = fp <= S - P

        iota = lax.iota(jnp.int32, L)
        one = jnp.full((L,), 1, jnp.int32)
        zero = jnp.full((L,), 0, jnp.int32)
        for j in range(TPW // L):
            v = row_v[pl.ds(loc + j * L, L)]
            mask_v[pl.ds(j * L, L)] = jnp.where(v != -100, one, zero)
        pltpu.sync_copy(mask_v, mask_hbm.at[b, pl.ds(loc, TPW)])

        o_pending = [False] * NCHUNK
        for k in range(NCHUNK):
            gd[k % NBUF].wait()
            fire_o(k)
            o_pending[k] = True
            nk = k + DEPTH
            if nk < NCHUNK:
                j = nk - NBUF
                if j >= 0:
                    od[j % NBUF].wait()
                    o_pending[j] = False
                fire_g(nk)
        for k in range(NCHUNK):
            if o_pending[k]:
                od[k % NBUF].wait()

        plsc.subcore_barrier()

        @pl.when(valid)
        def _image_overwrite():
            for h in range(IPW // L):
                pltpu.sync_copy(
                    img_hbm.at[b, pl.ds(kw * IPW + h * L, L)], ibuf)
                sidx_v[...] = fpv + kw * IPW + h * L + iota
                pltpu.async_copy(ibuf, out_hbm.at[b].at[sidx_v], sem).wait()

    return body


def kernel(input_ids, image_features, image_token_id, W):
    B, S = input_ids.shape
    _, P, H = image_features.shape
    V = W.shape[0]

    ids = input_ids.astype(jnp.int32)
    tid = jnp.full((16,), image_token_id, dtype=jnp.int32)

    sc = _build_sc_kernel(B, S, P, H, V)
    out, mask = sc(ids, image_features, tid, W)
    return out, mask
